# baseline (speedup 1.0000x reference)
import jax
import jax.numpy as jnp
from jax import lax
from jax.experimental import pallas as pl
from jax.experimental.pallas import tpu as pltpu
from jax.experimental.pallas import tpu_sc as plsc

N = 10000
E = 320000
D = 128
DE = 4
H = 128
NPG = 100

NC = 2
NS = 16
NPT = 640
NPAD = NS * NPT
CH = 256
NCHK = 40
EPT = CH * NCHK
EPAD = EPT * NS * NC
NEG = -1e30

_f32 = jnp.float32
_PREC = lax.Precision.HIGHEST


def _edge_chain_tc(e2, w1, b1, a1, w2, b2, a2, w3, b3, a3):
    e32 = E // 32
    blk = 1000

    def body(e_ref, w1_ref, b1_ref, a1_ref, w2_ref, b2_ref, a2_ref,
             w3_ref, b3_ref, a3_ref, o1_ref, o2_ref, o3_ref,
             m1_ref, m2_ref, m3_ref):
        i = pl.program_id(0)
        xv = e_ref[...]
        xv = jnp.where(jnp.isnan(xv), 0.0, xv)
        ea1 = jnp.dot(xv, w1_ref[...], preferred_element_type=_f32, precision=_PREC) + b1_ref[...]
        o1 = jnp.dot(ea1, a1_ref[...], preferred_element_type=_f32, precision=_PREC)
        o1_ref[...] = o1
        ea2 = jnp.dot(ea1, w2_ref[...], preferred_element_type=_f32, precision=_PREC) + b2_ref[...]
        o2 = jnp.dot(ea2, a2_ref[...], preferred_element_type=_f32, precision=_PREC)
        o2_ref[...] = o2
        ea3 = jnp.dot(ea2, w3_ref[...], preferred_element_type=_f32, precision=_PREC) + b3_ref[...]
        o3 = jnp.dot(ea3, a3_ref[...], preferred_element_type=_f32, precision=_PREC)
        o3_ref[...] = o3
        for o, m_ref in ((o1, m1_ref), (o2, m2_ref), (o3, m3_ref)):
            cur = jnp.full((1, 128), jnp.max(o), _f32)

            @pl.when(i == 0)
            def _():
                m_ref[...] = cur

            @pl.when(i > 0)
            def _():
                m_ref[...] = jnp.maximum(m_ref[...], cur)

    espec = pl.BlockSpec((blk, 128), lambda i: (i, 0))
    wspec = pl.BlockSpec((128, 128), lambda i: (0, 0))
    bspec = pl.BlockSpec((1, 128), lambda i: (0, 0))
    aspec = pl.BlockSpec((128, 32), lambda i: (0, 0))
    ospec = pl.BlockSpec((blk, 32), lambda i: (i, 0))
    mspec = pl.BlockSpec((1, 128), lambda i: (0, 0))
    return pl.pallas_call(
        body,
        grid=(e32 // blk,),
        in_specs=[espec, wspec, bspec, aspec, wspec, bspec, aspec,
                  wspec, bspec, aspec],
        out_specs=[ospec, ospec, ospec, mspec, mspec, mspec],
        out_shape=[jax.ShapeDtypeStruct((e32, 32), _f32)] * 3
        + [jax.ShapeDtypeStruct((1, 128), _f32)] * 3,
    )(e2, w1, b1, a1, w2, b2, a2, w3, b3, a3)


_BLK = 1024
_xspec = pl.BlockSpec((_BLK, 128), lambda i: (i, 0))
_wspec = pl.BlockSpec((128, 128), lambda i: (0, 0))
_bspec = pl.BlockSpec((1, 128), lambda i: (0, 0))
_vspec = pl.BlockSpec((_BLK,), lambda i: (i,))
_mspec = pl.BlockSpec((1, 128), lambda i: (0, 0))


def _node_tc(a0, a1, d0, d1, w, b, ad, as_, rflag):
    def body(a0_ref, a1_ref, d0_ref, d1_ref, w_ref, b_ref, ad_ref, as_ref,
             rf_ref, h_ref, hd_ref, hs_ref, mxd_ref, mxs_ref):
        i = pl.program_id(0)
        d = d0_ref[...] + d1_ref[...]
        inv = jnp.where(d > 0.0, 1.0 / d, 0.0)
        xv = (a0_ref[...] + a1_ref[...]) * inv[:, None]
        xv = jnp.where(rf_ref[...] > 0.0, jnp.maximum(xv, 0.0), xv)
        h = jnp.dot(xv, w_ref[...], preferred_element_type=_f32, precision=_PREC)
        h = h + b_ref[...]
        h_ref[...] = h
        hdv = jnp.sum(h * ad_ref[...], axis=1)
        hsv = jnp.sum(h * as_ref[...], axis=1)
        hd_ref[...] = hdv
        hs_ref[...] = hsv
        curd = jnp.full((1, 128), jnp.max(hdv), _f32)
        curs = jnp.full((1, 128), jnp.max(hsv), _f32)

        @pl.when(i == 0)
        def _():
            mxd_ref[...] = curd
            mxs_ref[...] = curs

        @pl.when(i > 0)
        def _():
            mxd_ref[...] = jnp.maximum(mxd_ref[...], curd)
            mxs_ref[...] = jnp.maximum(mxs_ref[...], curs)

    return pl.pallas_call(
        body,
        grid=(NPAD // _BLK,),
        in_specs=[_xspec, _xspec, _vspec, _vspec, _wspec, _bspec, _bspec,
                  _bspec, pl.BlockSpec((1, 1), lambda i: (0, 0))],
        out_specs=[_xspec, _vspec, _vspec, _mspec, _mspec],
        out_shape=[
            jax.ShapeDtypeStruct((NPAD, 128), _f32),
            jax.ShapeDtypeStruct((NPAD,), _f32),
            jax.ShapeDtypeStruct((NPAD,), _f32),
            jax.ShapeDtypeStruct((1, 128), _f32),
            jax.ShapeDtypeStruct((1, 128), _f32),
        ],
    )(a0, a1, d0, d1, w, b.reshape(1, 128), ad, as_, rflag)


def _final_tc(a0, a1, d0, d1, wl, bl):
    def body(a0_ref, a1_ref, d0_ref, d1_ref, w_ref, b_ref, o_ref):
        d = d0_ref[...] + d1_ref[...]
        inv = jnp.where(d > 0.0, 1.0 / d, 0.0)
        xv = (a0_ref[...] + a1_ref[...]) * inv[:, None]
        o_ref[...] = jnp.dot(xv, w_ref[...], preferred_element_type=_f32, precision=_PREC) + b_ref[...]

    return pl.pallas_call(
        body,
        grid=(NPAD // _BLK,),
        in_specs=[_xspec, _xspec, _vspec, _vspec,
                  pl.BlockSpec((128, 1), lambda i: (0, 0)),
                  pl.BlockSpec((1, 1), lambda i: (0, 0))],
        out_specs=pl.BlockSpec((_BLK, 1), lambda i: (i, 0)),
        out_shape=jax.ShapeDtypeStruct((NPAD, 1), _f32),
    )(a0, a1, d0, d1, wl, bl)


NF = 16
NP = H // NF


def _sc_layer_body(h8_hbm, hd_hbm, hs_hbm, mc_hbm, et_hbm, src_hbm, dst_hbm,
                   z2_hbm, z1_hbm, a0_hbm, a1_hbm, d0_hbm, d1_hbm,
                   hd_v, hs_v, src_v, src8_v, dst_v, et_v, rows_v, mbuf_v,
                   acc_s, den_s, gsem0, gsem1):
    c = lax.axis_index("c")
    s = lax.axis_index("s")
    base = (c * NS + s) * EPT

    pltpu.sync_copy(hd_hbm, hd_v)
    pltpu.sync_copy(hs_hbm, hs_v)
    for jj in range(NCHK):
        esl = pl.ds(base + jj * CH, CH)
        pltpu.sync_copy(src_hbm.at[esl], src_v.at[jj])
        pltpu.sync_copy(dst_hbm.at[esl], dst_v.at[jj])
        pltpu.sync_copy(et_hbm.at[esl], et_v.at[jj])
    pltpu.sync_copy(mc_hbm, mbuf_v)
    pltpu.sync_copy(z1_hbm, den_s.at[pl.ds(s * NPT, NPT)])
    plsc.subcore_barrier()

    mvec = mbuf_v[pl.ds(0, 16)] + mbuf_v[pl.ds(128, 16)] + mbuf_v[pl.ds(256, 16)]

    @pl.loop(0, NCHK)
    def _(j):
        @pl.loop(0, CH // 16)
        def _(u):
            sl = pl.ds(u * 16, 16)
            di = dst_v[j, sl]
            si = src_v[j, sl]
            l = (plsc.load_gather(hd_v, [di])
                 + plsc.load_gather(hs_v, [si]) + et_v[j, sl])
            l = jnp.where(l > 0.0, l, l * 0.2)
            et_v[j, sl] = jnp.exp(l - mvec)

    @pl.loop(0, NCHK)
    def _(j):
        pltpu.sync_copy(et_v.at[j], den_s.at[dst_v.at[j]], add=True)

    plsc.subcore_barrier()
    osl = pl.ds(s * NPT, NPT)

    @pl.when(c == 0)
    def _():
        pltpu.sync_copy(den_s.at[osl], d0_hbm.at[osl])

    @pl.when(c == 1)
    def _():
        pltpu.sync_copy(den_s.at[osl], d1_hbm.at[osl])

    gsems = (gsem0, gsem1)

    def gcopy(j, b):
        return pltpu.make_async_copy(
            h8_hbm.at[src8_v.at[j]], rows_v.at[b], gsems[b])

    @pl.loop(0, NP)
    def _(p):
        @pl.loop(0, NCHK)
        def _(j):
            @pl.loop(0, CH // 16)
            def _(u):
                sl = pl.ds(u * 16, 16)
                src8_v[j, sl] = src_v[j, sl] * NP + p

        pltpu.sync_copy(z2_hbm, acc_s.at[pl.ds(s * NPT, NPT)])
        plsc.subcore_barrier()

        gcopy(0, 0).start()

        def pair(q, _):
            for b in range(2):
                j = q * 2 + b

                @pl.when(j + 1 < NCHK)
                def _():
                    gcopy(j + 1, 1 - b).start()

                gcopy(j, b).wait()

                @pl.loop(0, CH // 16)
                def _(u):
                    ex16 = et_v[j, pl.ds(u * 16, 16)]
                    for t in range(16):
                        g = ex16[t]
                        r = u * 16 + t
                        rows_v[b, r, :] = rows_v[b, r, :] * g

                pltpu.sync_copy(rows_v.at[b], acc_s.at[dst_v.at[j]],
                                add=True)
            return 0

        lax.fori_loop(0, NCHK // 2, pair, 0)
        plsc.subcore_barrier()

        @pl.when(c == 0)
        def _():
            pltpu.sync_copy(acc_s.at[osl], a0_hbm.at[p, osl])

        @pl.when(c == 1)
        def _():
            pltpu.sync_copy(acc_s.at[osl], a1_hbm.at[p, osl])


_sc_layer = pl.kernel(
    _sc_layer_body,
    out_type=(jax.ShapeDtypeStruct((NP, NPAD, NF), _f32),
              jax.ShapeDtypeStruct((NP, NPAD, NF), _f32),
              jax.ShapeDtypeStruct((NPAD,), _f32),
              jax.ShapeDtypeStruct((NPAD,), _f32)),
    mesh=plsc.VectorSubcoreMesh(core_axis_name="c", subcore_axis_name="s"),
    compiler_params=pltpu.CompilerParams(needs_layout_passes=False,
                                         use_tc_tiling_on_sc=False),
    scratch_types=[
        pltpu.VMEM((NPAD,), _f32),
        pltpu.VMEM((NPAD,), _f32),
        pltpu.VMEM((NCHK, CH), jnp.int32),
        pltpu.VMEM((NCHK, CH), jnp.int32),
        pltpu.VMEM((NCHK, CH), jnp.int32),
        pltpu.VMEM((NCHK, CH), _f32),
        pltpu.VMEM((2, CH, NF), _f32),
        pltpu.VMEM((384,), _f32),
        pltpu.VMEM_SHARED((NPAD, NF), _f32),
        pltpu.VMEM_SHARED((NPAD,), _f32),
        pltpu.SemaphoreType.DMA,
        pltpu.SemaphoreType.DMA,
    ],
)


def kernel(x, edge_index, edge_attr, batch, Wn1, bn1, We1, be1, att1,
           Wn2, bn2, We2, be2, att2, Wn3, bn3, We3, be3, att3, Wl, bl):
    del batch

    src = edge_index[0]
    dst = edge_index[1]
    pad_idx = (jnp.arange(EPAD - E, dtype=jnp.int32) % N)
    src1 = jnp.concatenate([src, pad_idx])
    dst1 = jnp.concatenate([dst, pad_idx])

    e2 = edge_attr.reshape(E // 32, 128)
    eye = jnp.eye(32, dtype=_f32)

    def kron_w(we, be, att):
        wk = jnp.kron(eye, we)
        bk = jnp.tile(be, 32).reshape(1, 128)
        ak = jnp.kron(eye, att[2 * H:].reshape(DE, 1))
        return wk, bk, ak

    w1k, b1k, a1k = kron_w(We1, be1, att1)
    w2k, b2k, a2k = kron_w(We2, be2, att2)
    w3k, b3k, a3k = kron_w(We3, be3, att3)

    et1, et2, et3, me1, me2, me3 = _edge_chain_tc(
        e2, w1k, b1k, a1k, w2k, b2k, a2k, w3k, b3k, a3k)

    neg = jnp.full((EPAD - E,), NEG, _f32)

    def pack_et(et):
        return jnp.concatenate([et.reshape(E), neg])

    et1p, et2p, et3p = pack_et(et1), pack_et(et2), pack_et(et3)

    xpad = jnp.pad(x, ((0, NPAD - N), (0, 0)))
    z2 = jnp.zeros((NPT, NF), _f32)
    z1 = jnp.zeros((NPT,), _f32)

    def att_parts(att):
        return att[:H].reshape(1, 128), att[H:2 * H].reshape(1, 128)

    ad1, as1 = att_parts(att1)
    ad2, as2 = att_parts(att2)
    ad3, as3 = att_parts(att3)

    wn_s = jnp.stack([Wn1, Wn2, Wn3])
    bn_s = jnp.stack([bn1, bn2, bn3])
    ad_s = jnp.stack([ad1, ad2, ad3])
    as_s = jnp.stack([as1, as2, as3])
    rf_s = jnp.asarray([0.0, 1.0, 1.0], _f32).reshape(3, 1, 1)
    et_s = jnp.stack([et1p, et2p, et3p])
    me_s = jnp.stack([me1, me2, me3])

    def layer(carry, xs):
        a0, a1_, d0, d1 = carry
        wn, bn, ad, as_, rf, etp, mce = xs
        h, hd, hs, mxd, mxs = _node_tc(a0, a1_, d0, d1, wn, bn, ad, as_, rf)
        mc = jnp.concatenate([mxd, mxs, mce], axis=1).reshape(384)
        h8 = h.reshape(NPAD * NP, NF)
        a0_8, a1_8, d0, d1 = _sc_layer(h8, hd, hs, mc, etp, src1, dst1,
                                       z2, z1)
        a0 = a0_8.transpose((1, 0, 2)).reshape(NPAD, 128)
        a1_ = a1_8.transpose((1, 0, 2)).reshape(NPAD, 128)
        return (a0, a1_, d0, d1), None

    zeros = jnp.zeros((NPAD, 128), _f32)
    halves = jnp.full((NPAD,), 0.5, _f32)
    init = (xpad, zeros, halves, halves)
    (a0, a1_, d0, d1), _ = lax.scan(
        layer, init, (wn_s, bn_s, ad_s, as_s, rf_s, et_s, me_s))

    y = _final_tc(a0, a1_, d0, d1, Wl, bl.reshape(1, 1))
    return y[(NPG - 1) // 2:N:NPG]


# speedup vs baseline: 1.3541x; 1.3541x over previous
import jax
import jax.numpy as jnp
from jax import lax
from jax.experimental import pallas as pl
from jax.experimental.pallas import tpu as pltpu
from jax.experimental.pallas import tpu_sc as plsc

N = 10000
E = 320000
D = 128
DE = 4
H = 128
NPG = 100

NC = 2
NS = 16
NPT = 640
NPAD = NS * NPT
CH = 256
NCHK = 40
EPT = CH * NCHK
EPAD = EPT * NS * NC
NEG = -1e30

_f32 = jnp.float32
_PREC = lax.Precision.HIGHEST


def _edge_chain_tc(e2, w1, b1, a1, w2, b2, a2, w3, b3, a3):
    e32 = E // 32
    blk = 1000

    def body(e_ref, w1_ref, b1_ref, a1_ref, w2_ref, b2_ref, a2_ref,
             w3_ref, b3_ref, a3_ref, o1_ref, o2_ref, o3_ref,
             m1_ref, m2_ref, m3_ref):
        i = pl.program_id(0)
        xv = e_ref[...]
        xv = jnp.where(jnp.isnan(xv), 0.0, xv)
        ea1 = jnp.dot(xv, w1_ref[...], preferred_element_type=_f32, precision=_PREC) + b1_ref[...]
        o1 = jnp.dot(ea1, a1_ref[...], preferred_element_type=_f32, precision=_PREC)
        o1_ref[...] = o1
        ea2 = jnp.dot(ea1, w2_ref[...], preferred_element_type=_f32, precision=_PREC) + b2_ref[...]
        o2 = jnp.dot(ea2, a2_ref[...], preferred_element_type=_f32, precision=_PREC)
        o2_ref[...] = o2
        ea3 = jnp.dot(ea2, w3_ref[...], preferred_element_type=_f32, precision=_PREC) + b3_ref[...]
        o3 = jnp.dot(ea3, a3_ref[...], preferred_element_type=_f32, precision=_PREC)
        o3_ref[...] = o3
        for o, m_ref in ((o1, m1_ref), (o2, m2_ref), (o3, m3_ref)):
            cur = jnp.full((1, 128), jnp.max(o), _f32)

            @pl.when(i == 0)
            def _():
                m_ref[...] = cur

            @pl.when(i > 0)
            def _():
                m_ref[...] = jnp.maximum(m_ref[...], cur)

    espec = pl.BlockSpec((blk, 128), lambda i: (i, 0))
    wspec = pl.BlockSpec((128, 128), lambda i: (0, 0))
    bspec = pl.BlockSpec((1, 128), lambda i: (0, 0))
    aspec = pl.BlockSpec((128, 32), lambda i: (0, 0))
    ospec = pl.BlockSpec((blk, 32), lambda i: (i, 0))
    mspec = pl.BlockSpec((1, 128), lambda i: (0, 0))
    return pl.pallas_call(
        body,
        grid=(e32 // blk,),
        in_specs=[espec, wspec, bspec, aspec, wspec, bspec, aspec,
                  wspec, bspec, aspec],
        out_specs=[ospec, ospec, ospec, mspec, mspec, mspec],
        out_shape=[jax.ShapeDtypeStruct((e32, 32), _f32)] * 3
        + [jax.ShapeDtypeStruct((1, 128), _f32)] * 3,
    )(e2, w1, b1, a1, w2, b2, a2, w3, b3, a3)


_BLK = 1024
_xspec = pl.BlockSpec((_BLK, 128), lambda i: (i, 0))
_wspec = pl.BlockSpec((128, 128), lambda i: (0, 0))
_bspec = pl.BlockSpec((1, 128), lambda i: (0, 0))
_vspec = pl.BlockSpec((_BLK,), lambda i: (i,))
_mspec = pl.BlockSpec((1, 128), lambda i: (0, 0))


def _node_tc(a0, a1, d0, d1, w, b, ad, as_, rflag):
    def body(a0_ref, a1_ref, d0_ref, d1_ref, w_ref, b_ref, ad_ref, as_ref,
             rf_ref, h_ref, hd_ref, hs_ref, mxd_ref, mxs_ref):
        i = pl.program_id(0)
        d = d0_ref[...] + d1_ref[...]
        inv = jnp.where(d > 0.0, 1.0 / d, 0.0)
        xv = (a0_ref[...] + a1_ref[...]) * inv[:, None]
        xv = jnp.where(rf_ref[...] > 0.0, jnp.maximum(xv, 0.0), xv)
        h = jnp.dot(xv, w_ref[...], preferred_element_type=_f32, precision=_PREC)
        h = h + b_ref[...]
        h_ref[...] = h
        hdv = jnp.sum(h * ad_ref[...], axis=1)
        hsv = jnp.sum(h * as_ref[...], axis=1)
        hd_ref[...] = hdv
        hs_ref[...] = hsv
        curd = jnp.full((1, 128), jnp.max(hdv), _f32)
        curs = jnp.full((1, 128), jnp.max(hsv), _f32)

        @pl.when(i == 0)
        def _():
            mxd_ref[...] = curd
            mxs_ref[...] = curs

        @pl.when(i > 0)
        def _():
            mxd_ref[...] = jnp.maximum(mxd_ref[...], curd)
            mxs_ref[...] = jnp.maximum(mxs_ref[...], curs)

    return pl.pallas_call(
        body,
        grid=(NPAD // _BLK,),
        in_specs=[_xspec, _xspec, _vspec, _vspec, _wspec, _bspec, _bspec,
                  _bspec, pl.BlockSpec((1, 1), lambda i: (0, 0))],
        out_specs=[_xspec, _vspec, _vspec, _mspec, _mspec],
        out_shape=[
            jax.ShapeDtypeStruct((NPAD, 128), _f32),
            jax.ShapeDtypeStruct((NPAD,), _f32),
            jax.ShapeDtypeStruct((NPAD,), _f32),
            jax.ShapeDtypeStruct((1, 128), _f32),
            jax.ShapeDtypeStruct((1, 128), _f32),
        ],
    )(a0, a1, d0, d1, w, b.reshape(1, 128), ad, as_, rflag)


def _final_tc(a0, a1, d0, d1, wl, bl):
    def body(a0_ref, a1_ref, d0_ref, d1_ref, w_ref, b_ref, o_ref):
        d = d0_ref[...] + d1_ref[...]
        inv = jnp.where(d > 0.0, 1.0 / d, 0.0)
        xv = (a0_ref[...] + a1_ref[...]) * inv[:, None]
        o_ref[...] = jnp.dot(xv, w_ref[...], preferred_element_type=_f32, precision=_PREC) + b_ref[...]

    return pl.pallas_call(
        body,
        grid=(NPAD // _BLK,),
        in_specs=[_xspec, _xspec, _vspec, _vspec,
                  pl.BlockSpec((128, 1), lambda i: (0, 0)),
                  pl.BlockSpec((1, 1), lambda i: (0, 0))],
        out_specs=pl.BlockSpec((_BLK, 1), lambda i: (i, 0)),
        out_shape=jax.ShapeDtypeStruct((NPAD, 1), _f32),
    )(a0, a1, d0, d1, wl, bl)


NF = 16
NP = H // NF


def _sc_layer_body(h8_hbm, hd_hbm, hs_hbm, mc_hbm, et_hbm, src_hbm, dst_hbm,
                   z2_hbm, z1_hbm, a0_hbm, a1_hbm, d0_hbm, d1_hbm,
                   hd_v, hs_v, src_v, src8_v, dst_v, et_v, rows_v, mbuf_v,
                   acc_s, den_s, gsem0, gsem1):
    c = lax.axis_index("c")
    s = lax.axis_index("s")
    tid = c * NS + s

    pltpu.sync_copy(hd_hbm, hd_v)
    pltpu.sync_copy(hs_hbm, hs_v)
    pltpu.sync_copy(src_hbm.at[tid], src_v)
    pltpu.sync_copy(dst_hbm.at[tid], dst_v)
    pltpu.sync_copy(et_hbm.at[tid], et_v)
    pltpu.sync_copy(mc_hbm, mbuf_v)
    pltpu.sync_copy(z1_hbm, den_s.at[pl.ds(s * NPT, NPT)])
    plsc.subcore_barrier()

    mvec = mbuf_v[pl.ds(0, 16)] + mbuf_v[pl.ds(128, 16)] + mbuf_v[pl.ds(256, 16)]

    @pl.loop(0, NCHK)
    def _(j):
        @pl.loop(0, CH // 16)
        def _(u):
            sl = pl.ds(u * 16, 16)
            di = dst_v[j, sl]
            si = src_v[j, sl]
            l = (plsc.load_gather(hd_v, [di])
                 + plsc.load_gather(hs_v, [si]) + et_v[j, sl])
            l = jnp.where(l > 0.0, l, l * 0.2)
            et_v[j, sl] = jnp.exp(l - mvec)

    @pl.loop(0, NCHK)
    def _(j):
        pltpu.sync_copy(et_v.at[j], den_s.at[dst_v.at[j]], add=True)

    plsc.subcore_barrier()
    osl = pl.ds(s * NPT, NPT)

    @pl.when(c == 0)
    def _():
        pltpu.sync_copy(den_s.at[osl], d0_hbm.at[osl])

    @pl.when(c == 1)
    def _():
        pltpu.sync_copy(den_s.at[osl], d1_hbm.at[osl])

    gsems = (gsem0, gsem1)

    def gcopy(j, b):
        return pltpu.make_async_copy(
            h8_hbm.at[src8_v.at[j]], rows_v.at[b], gsems[b])

    @pl.loop(0, NP)
    def _(p):
        @pl.loop(0, NCHK)
        def _(j):
            @pl.loop(0, CH // 16)
            def _(u):
                sl = pl.ds(u * 16, 16)
                src8_v[j, sl] = src_v[j, sl] * NP + p

        pltpu.sync_copy(z2_hbm, acc_s.at[pl.ds(s * NPT, NPT)])
        plsc.subcore_barrier()

        gcopy(0, 0).start()

        def pair(q, _):
            for b in range(2):
                j = q * 2 + b

                @pl.when(j + 1 < NCHK)
                def _():
                    gcopy(j + 1, 1 - b).start()

                gcopy(j, b).wait()

                @pl.loop(0, CH // 16)
                def _(u):
                    ex16 = et_v[j, pl.ds(u * 16, 16)]
                    for t in range(16):
                        g = ex16[t]
                        r = u * 16 + t
                        rows_v[b, r, :] = rows_v[b, r, :] * g

                pltpu.sync_copy(rows_v.at[b], acc_s.at[dst_v.at[j]],
                                add=True)
            return 0

        lax.fori_loop(0, NCHK // 2, pair, 0)
        plsc.subcore_barrier()

        fsl = pl.ds(p * NF, NF)

        @pl.when(c == 0)
        def _():
            pltpu.sync_copy(acc_s.at[osl], a0_hbm.at[osl, fsl])

        @pl.when(c == 1)
        def _():
            pltpu.sync_copy(acc_s.at[osl], a1_hbm.at[osl, fsl])


_sc_layer = pl.kernel(
    _sc_layer_body,
    out_type=(jax.ShapeDtypeStruct((NPAD, 128), _f32),
              jax.ShapeDtypeStruct((NPAD, 128), _f32),
              jax.ShapeDtypeStruct((NPAD,), _f32),
              jax.ShapeDtypeStruct((NPAD,), _f32)),
    mesh=plsc.VectorSubcoreMesh(core_axis_name="c", subcore_axis_name="s"),
    compiler_params=pltpu.CompilerParams(needs_layout_passes=False,
                                         use_tc_tiling_on_sc=False),
    scratch_types=[
        pltpu.VMEM((NPAD,), _f32),
        pltpu.VMEM((NPAD,), _f32),
        pltpu.VMEM((NCHK, CH), jnp.int32),
        pltpu.VMEM((NCHK, CH), jnp.int32),
        pltpu.VMEM((NCHK, CH), jnp.int32),
        pltpu.VMEM((NCHK, CH), _f32),
        pltpu.VMEM((2, CH, NF), _f32),
        pltpu.VMEM((384,), _f32),
        pltpu.VMEM_SHARED((NPAD, NF), _f32),
        pltpu.VMEM_SHARED((NPAD,), _f32),
        pltpu.SemaphoreType.DMA,
        pltpu.SemaphoreType.DMA,
    ],
)


def kernel(x, edge_index, edge_attr, batch, Wn1, bn1, We1, be1, att1,
           Wn2, bn2, We2, be2, att2, Wn3, bn3, We3, be3, att3, Wl, bl):
    del batch

    src = edge_index[0]
    dst = edge_index[1]
    pad_idx = (jnp.arange(EPAD - E, dtype=jnp.int32) % N)
    src1 = jnp.concatenate([src, pad_idx]).reshape(NC * NS, NCHK, CH)
    dst1 = jnp.concatenate([dst, pad_idx]).reshape(NC * NS, NCHK, CH)

    e2 = edge_attr.reshape(E // 32, 128)
    eye = jnp.eye(32, dtype=_f32)

    def kron_w(we, be, att):
        wk = jnp.kron(eye, we)
        bk = jnp.tile(be, 32).reshape(1, 128)
        ak = jnp.kron(eye, att[2 * H:].reshape(DE, 1))
        return wk, bk, ak

    w1k, b1k, a1k = kron_w(We1, be1, att1)
    w2k, b2k, a2k = kron_w(We2, be2, att2)
    w3k, b3k, a3k = kron_w(We3, be3, att3)

    et1, et2, et3, me1, me2, me3 = _edge_chain_tc(
        e2, w1k, b1k, a1k, w2k, b2k, a2k, w3k, b3k, a3k)

    neg = jnp.full((EPAD - E,), NEG, _f32)

    def pack_et(et):
        return jnp.concatenate([et.reshape(E), neg]).reshape(NC * NS, NCHK, CH)

    et1p, et2p, et3p = pack_et(et1), pack_et(et2), pack_et(et3)

    xpad = jnp.pad(x, ((0, NPAD - N), (0, 0)))
    z2 = jnp.zeros((NPT, NF), _f32)
    z1 = jnp.zeros((NPT,), _f32)

    def att_parts(att):
        return att[:H].reshape(1, 128), att[H:2 * H].reshape(1, 128)

    ad1, as1 = att_parts(att1)
    ad2, as2 = att_parts(att2)
    ad3, as3 = att_parts(att3)

    wn_s = jnp.stack([Wn1, Wn2, Wn3])
    bn_s = jnp.stack([bn1, bn2, bn3])
    ad_s = jnp.stack([ad1, ad2, ad3])
    as_s = jnp.stack([as1, as2, as3])
    rf_s = jnp.asarray([0.0, 1.0, 1.0], _f32).reshape(3, 1, 1)
    et_s = jnp.stack([et1p, et2p, et3p])
    me_s = jnp.stack([me1, me2, me3])

    def layer(carry, xs):
        a0, a1_, d0, d1 = carry
        wn, bn, ad, as_, rf, etp, mce = xs
        h, hd, hs, mxd, mxs = _node_tc(a0, a1_, d0, d1, wn, bn, ad, as_, rf)
        mc = jnp.concatenate([mxd, mxs, mce], axis=1).reshape(384)
        h8 = h.reshape(NPAD * NP, NF)
        a0, a1_, d0, d1 = _sc_layer(h8, hd, hs, mc, etp, src1, dst1,
                                    z2, z1)
        return (a0, a1_, d0, d1), None

    zeros = jnp.zeros((NPAD, 128), _f32)
    halves = jnp.full((NPAD,), 0.5, _f32)
    init = (xpad, zeros, halves, halves)
    (a0, a1_, d0, d1), _ = lax.scan(
        layer, init, (wn_s, bn_s, ad_s, as_s, rf_s, et_s, me_s))

    y = _final_tc(a0, a1_, d0, d1, Wl, bl.reshape(1, 1))
    return y[(NPG - 1) // 2:N:NPG]


# speedup vs baseline: 1.4555x; 1.0749x over previous
import jax
import jax.numpy as jnp
from jax import lax
from jax.experimental import pallas as pl
from jax.experimental.pallas import tpu as pltpu
from jax.experimental.pallas import tpu_sc as plsc

N = 10000
E = 320000
D = 128
DE = 4
H = 128
NPG = 100

NC = 2
NS = 16
NPT = 640
NPAD = NS * NPT
CH = 512
NCHK = 20
EPT = CH * NCHK
EPAD = EPT * NS * NC
NEG = -1e30

_f32 = jnp.float32
_PREC = lax.Precision.HIGHEST


def _edge_chain_tc(e2, w1, b1, a1, w2, b2, a2, w3, b3, a3):
    e32 = E // 32
    blk = 1000

    def body(e_ref, w1_ref, b1_ref, a1_ref, w2_ref, b2_ref, a2_ref,
             w3_ref, b3_ref, a3_ref, o1_ref, o2_ref, o3_ref,
             m1_ref, m2_ref, m3_ref):
        i = pl.program_id(0)
        xv = e_ref[...]
        xv = jnp.where(jnp.isnan(xv), 0.0, xv)
        ea1 = jnp.dot(xv, w1_ref[...], preferred_element_type=_f32, precision=_PREC) + b1_ref[...]
        o1 = jnp.dot(ea1, a1_ref[...], preferred_element_type=_f32, precision=_PREC)
        o1_ref[...] = o1
        ea2 = jnp.dot(ea1, w2_ref[...], preferred_element_type=_f32, precision=_PREC) + b2_ref[...]
        o2 = jnp.dot(ea2, a2_ref[...], preferred_element_type=_f32, precision=_PREC)
        o2_ref[...] = o2
        ea3 = jnp.dot(ea2, w3_ref[...], preferred_element_type=_f32, precision=_PREC) + b3_ref[...]
        o3 = jnp.dot(ea3, a3_ref[...], preferred_element_type=_f32, precision=_PREC)
        o3_ref[...] = o3
        for o, m_ref in ((o1, m1_ref), (o2, m2_ref), (o3, m3_ref)):
            cur = jnp.full((1, 128), jnp.max(o), _f32)

            @pl.when(i == 0)
            def _():
                m_ref[...] = cur

            @pl.when(i > 0)
            def _():
                m_ref[...] = jnp.maximum(m_ref[...], cur)

    espec = pl.BlockSpec((blk, 128), lambda i: (i, 0))
    wspec = pl.BlockSpec((128, 128), lambda i: (0, 0))
    bspec = pl.BlockSpec((1, 128), lambda i: (0, 0))
    aspec = pl.BlockSpec((128, 32), lambda i: (0, 0))
    ospec = pl.BlockSpec((blk, 32), lambda i: (i, 0))
    mspec = pl.BlockSpec((1, 128), lambda i: (0, 0))
    return pl.pallas_call(
        body,
        grid=(e32 // blk,),
        in_specs=[espec, wspec, bspec, aspec, wspec, bspec, aspec,
                  wspec, bspec, aspec],
        out_specs=[ospec, ospec, ospec, mspec, mspec, mspec],
        out_shape=[jax.ShapeDtypeStruct((e32, 32), _f32)] * 3
        + [jax.ShapeDtypeStruct((1, 128), _f32)] * 3,
    )(e2, w1, b1, a1, w2, b2, a2, w3, b3, a3)


_BLK = 1024
_xspec = pl.BlockSpec((_BLK, 128), lambda i: (i, 0))
_wspec = pl.BlockSpec((128, 128), lambda i: (0, 0))
_bspec = pl.BlockSpec((1, 128), lambda i: (0, 0))
_vspec = pl.BlockSpec((_BLK,), lambda i: (i,))
_mspec = pl.BlockSpec((1, 128), lambda i: (0, 0))


def _node_tc(a0, a1, d0, d1, w, b, ad, as_, rflag):
    def body(a0_ref, a1_ref, d0_ref, d1_ref, w_ref, b_ref, ad_ref, as_ref,
             rf_ref, h_ref, hd_ref, hs_ref, mxd_ref, mxs_ref):
        i = pl.program_id(0)
        d = d0_ref[...] + d1_ref[...]
        inv = jnp.where(d > 0.0, 1.0 / d, 0.0)
        xv = (a0_ref[...] + a1_ref[...]) * inv[:, None]
        xv = jnp.where(rf_ref[...] > 0.0, jnp.maximum(xv, 0.0), xv)
        h = jnp.dot(xv, w_ref[...], preferred_element_type=_f32, precision=_PREC)
        h = h + b_ref[...]
        h_ref[...] = h
        hdv = jnp.sum(h * ad_ref[...], axis=1)
        hsv = jnp.sum(h * as_ref[...], axis=1)
        hd_ref[...] = hdv
        hs_ref[...] = hsv
        curd = jnp.full((1, 128), jnp.max(hdv), _f32)
        curs = jnp.full((1, 128), jnp.max(hsv), _f32)

        @pl.when(i == 0)
        def _():
            mxd_ref[...] = curd
            mxs_ref[...] = curs

        @pl.when(i > 0)
        def _():
            mxd_ref[...] = jnp.maximum(mxd_ref[...], curd)
            mxs_ref[...] = jnp.maximum(mxs_ref[...], curs)

    return pl.pallas_call(
        body,
        grid=(NPAD // _BLK,),
        in_specs=[_xspec, _xspec, _vspec, _vspec, _wspec, _bspec, _bspec,
                  _bspec, pl.BlockSpec((1, 1), lambda i: (0, 0))],
        out_specs=[_xspec, _vspec, _vspec, _mspec, _mspec],
        out_shape=[
            jax.ShapeDtypeStruct((NPAD, 128), _f32),
            jax.ShapeDtypeStruct((NPAD,), _f32),
            jax.ShapeDtypeStruct((NPAD,), _f32),
            jax.ShapeDtypeStruct((1, 128), _f32),
            jax.ShapeDtypeStruct((1, 128), _f32),
        ],
    )(a0, a1, d0, d1, w, b.reshape(1, 128), ad, as_, rflag)


def _final_tc(a0, a1, d0, d1, wl, bl):
    def body(a0_ref, a1_ref, d0_ref, d1_ref, w_ref, b_ref, o_ref):
        d = d0_ref[...] + d1_ref[...]
        inv = jnp.where(d > 0.0, 1.0 / d, 0.0)
        xv = (a0_ref[...] + a1_ref[...]) * inv[:, None]
        o_ref[...] = jnp.dot(xv, w_ref[...], preferred_element_type=_f32, precision=_PREC) + b_ref[...]

    return pl.pallas_call(
        body,
        grid=(NPAD // _BLK,),
        in_specs=[_xspec, _xspec, _vspec, _vspec,
                  pl.BlockSpec((128, 1), lambda i: (0, 0)),
                  pl.BlockSpec((1, 1), lambda i: (0, 0))],
        out_specs=pl.BlockSpec((_BLK, 1), lambda i: (i, 0)),
        out_shape=jax.ShapeDtypeStruct((NPAD, 1), _f32),
    )(a0, a1, d0, d1, wl, bl)


NF = 16
NP = H // NF


def _sc_layer_body(h8_hbm, hd_hbm, hs_hbm, mc_hbm, et_hbm, src_hbm, dst_hbm,
                   z2_hbm, z1_hbm, a0_hbm, a1_hbm, d0_hbm, d1_hbm,
                   hd_v, hs_v, src_v, src8_v, dst_v, et_v, rows_v, mbuf_v,
                   acc_s, den_s, gsem0, gsem1):
    c = lax.axis_index("c")
    s = lax.axis_index("s")
    tid = c * NS + s

    pltpu.sync_copy(hd_hbm, hd_v)
    pltpu.sync_copy(hs_hbm, hs_v)
    pltpu.sync_copy(src_hbm.at[tid], src_v)
    pltpu.sync_copy(dst_hbm.at[tid], dst_v)
    pltpu.sync_copy(et_hbm.at[tid], et_v)
    pltpu.sync_copy(mc_hbm, mbuf_v)
    pltpu.sync_copy(z1_hbm, den_s.at[pl.ds(s * NPT, NPT)])
    plsc.subcore_barrier()

    mvec = mbuf_v[pl.ds(0, 16)] + mbuf_v[pl.ds(128, 16)] + mbuf_v[pl.ds(256, 16)]

    @pl.loop(0, NCHK)
    def _(j):
        @pl.loop(0, CH // 16)
        def _(u):
            sl = pl.ds(u * 16, 16)
            di = dst_v[j, sl]
            si = src_v[j, sl]
            l = (plsc.load_gather(hd_v, [di])
                 + plsc.load_gather(hs_v, [si]) + et_v[j, sl])
            l = jnp.where(l > 0.0, l, l * 0.2)
            et_v[j, sl] = jnp.exp(l - mvec)

    @pl.loop(0, NCHK)
    def _(j):
        pltpu.sync_copy(et_v.at[j], den_s.at[dst_v.at[j]], add=True)

    plsc.subcore_barrier()
    osl = pl.ds(s * NPT, NPT)

    @pl.when(c == 0)
    def _():
        pltpu.sync_copy(den_s.at[osl], d0_hbm.at[osl])

    @pl.when(c == 1)
    def _():
        pltpu.sync_copy(den_s.at[osl], d1_hbm.at[osl])

    gsems = (gsem0, gsem1)

    def gcopy(j, b):
        return pltpu.make_async_copy(
            h8_hbm.at[src8_v.at[j]], rows_v.at[b], gsems[b])

    @pl.loop(0, NP)
    def _(p):
        @pl.loop(0, NCHK)
        def _(j):
            @pl.loop(0, CH // 16)
            def _(u):
                sl = pl.ds(u * 16, 16)
                src8_v[j, sl] = src_v[j, sl] * NP + p

        pltpu.sync_copy(z2_hbm, acc_s.at[pl.ds(s * NPT, NPT)])
        plsc.subcore_barrier()

        gcopy(0, 0).start()

        def pair(q, _):
            for b in range(2):
                j = q * 2 + b

                @pl.when(j + 1 < NCHK)
                def _():
                    gcopy(j + 1, 1 - b).start()

                gcopy(j, b).wait()

                @pl.loop(0, CH // 16)
                def _(u):
                    ex16 = et_v[j, pl.ds(u * 16, 16)]
                    for t in range(16):
                        g = ex16[t]
                        r = u * 16 + t
                        rows_v[b, r, :] = rows_v[b, r, :] * g

                pltpu.sync_copy(rows_v.at[b], acc_s.at[dst_v.at[j]],
                                add=True)
            return 0

        lax.fori_loop(0, NCHK // 2, pair, 0)
        plsc.subcore_barrier()

        fsl = pl.ds(p * NF, NF)

        @pl.when(c == 0)
        def _():
            pltpu.sync_copy(acc_s.at[osl], a0_hbm.at[osl, fsl])

        @pl.when(c == 1)
        def _():
            pltpu.sync_copy(acc_s.at[osl], a1_hbm.at[osl, fsl])


_sc_layer = pl.kernel(
    _sc_layer_body,
    out_type=(jax.ShapeDtypeStruct((NPAD, 128), _f32),
              jax.ShapeDtypeStruct((NPAD, 128), _f32),
              jax.ShapeDtypeStruct((NPAD,), _f32),
              jax.ShapeDtypeStruct((NPAD,), _f32)),
    mesh=plsc.VectorSubcoreMesh(core_axis_name="c", subcore_axis_name="s"),
    compiler_params=pltpu.CompilerParams(needs_layout_passes=False,
                                         use_tc_tiling_on_sc=False),
    scratch_types=[
        pltpu.VMEM((NPAD,), _f32),
        pltpu.VMEM((NPAD,), _f32),
        pltpu.VMEM((NCHK, CH), jnp.int32),
        pltpu.VMEM((NCHK, CH), jnp.int32),
        pltpu.VMEM((NCHK, CH), jnp.int32),
        pltpu.VMEM((NCHK, CH), _f32),
        pltpu.VMEM((2, CH, NF), _f32),
        pltpu.VMEM((384,), _f32),
        pltpu.VMEM_SHARED((NPAD, NF), _f32),
        pltpu.VMEM_SHARED((NPAD,), _f32),
        pltpu.SemaphoreType.DMA,
        pltpu.SemaphoreType.DMA,
    ],
)


def kernel(x, edge_index, edge_attr, batch, Wn1, bn1, We1, be1, att1,
           Wn2, bn2, We2, be2, att2, Wn3, bn3, We3, be3, att3, Wl, bl):
    del batch

    src = edge_index[0]
    dst = edge_index[1]
    pad_idx = (jnp.arange(EPAD - E, dtype=jnp.int32) % N)
    src1 = jnp.concatenate([src, pad_idx]).reshape(NC * NS, NCHK, CH)
    dst1 = jnp.concatenate([dst, pad_idx]).reshape(NC * NS, NCHK, CH)

    e2 = edge_attr.reshape(E // 32, 128)
    eye = jnp.eye(32, dtype=_f32)

    def kron_w(we, be, att):
        wk = jnp.kron(eye, we)
        bk = jnp.tile(be, 32).reshape(1, 128)
        ak = jnp.kron(eye, att[2 * H:].reshape(DE, 1))
        return wk, bk, ak

    w1k, b1k, a1k = kron_w(We1, be1, att1)
    w2k, b2k, a2k = kron_w(We2, be2, att2)
    w3k, b3k, a3k = kron_w(We3, be3, att3)

    et1, et2, et3, me1, me2, me3 = _edge_chain_tc(
        e2, w1k, b1k, a1k, w2k, b2k, a2k, w3k, b3k, a3k)

    neg = jnp.full((EPAD - E,), NEG, _f32)

    def pack_et(et):
        return jnp.concatenate([et.reshape(E), neg]).reshape(NC * NS, NCHK, CH)

    et1p, et2p, et3p = pack_et(et1), pack_et(et2), pack_et(et3)

    xpad = jnp.pad(x, ((0, NPAD - N), (0, 0)))
    z2 = jnp.zeros((NPT, NF), _f32)
    z1 = jnp.zeros((NPT,), _f32)

    def att_parts(att):
        return att[:H].reshape(1, 128), att[H:2 * H].reshape(1, 128)

    ad1, as1 = att_parts(att1)
    ad2, as2 = att_parts(att2)
    ad3, as3 = att_parts(att3)

    wn_s = jnp.stack([Wn1, Wn2, Wn3])
    bn_s = jnp.stack([bn1, bn2, bn3])
    ad_s = jnp.stack([ad1, ad2, ad3])
    as_s = jnp.stack([as1, as2, as3])
    rf_s = jnp.asarray([0.0, 1.0, 1.0], _f32).reshape(3, 1, 1)
    et_s = jnp.stack([et1p, et2p, et3p])
    me_s = jnp.stack([me1, me2, me3])

    def layer(carry, xs):
        a0, a1_, d0, d1 = carry
        wn, bn, ad, as_, rf, etp, mce = xs
        h, hd, hs, mxd, mxs = _node_tc(a0, a1_, d0, d1, wn, bn, ad, as_, rf)
        mc = jnp.concatenate([mxd, mxs, mce], axis=1).reshape(384)
        h8 = h.reshape(NPAD * NP, NF)
        a0, a1_, d0, d1 = _sc_layer(h8, hd, hs, mc, etp, src1, dst1,
                                    z2, z1)
        return (a0, a1_, d0, d1), None

    zeros = jnp.zeros((NPAD, 128), _f32)
    halves = jnp.full((NPAD,), 0.5, _f32)
    init = (xpad, zeros, halves, halves)
    (a0, a1_, d0, d1), _ = lax.scan(
        layer, init, (wn_s, bn_s, ad_s, as_s, rf_s, et_s, me_s))

    y = _final_tc(a0, a1_, d0, d1, Wl, bl.reshape(1, 1))
    return y[(NPG - 1) // 2:N:NPG]
